# baseline (speedup 1.0000x reference)
import functools
import jax
import jax.numpy as jnp
from jax import lax
from jax.experimental import pallas as pl
from jax.experimental.pallas import tpu as pltpu
from jax.experimental.pallas import tpu_sc as plsc

N = 10000
E = 320000
D = 128
H1 = 32
H2 = 16
DW = 8

NC = 2
NS = 16
NW = NC * NS
CP = 128
KP = 80
EPAD = NW * KP * CP
NP = N + 8
NWR = 10
RW = N // NWR
R = 1000
WAVE = 10
NWAVES = KP // WAVE
WAVE_D = 20

_mesh = plsc.VectorSubcoreMesh(core_axis_name="c", subcore_axis_name="s")


def _prep_edges(edge_index):
    pad = EPAD - E
    srcp = jnp.concatenate(
        [edge_index[0], jnp.zeros((pad,), jnp.int32)])
    dstp = jnp.concatenate(
        [edge_index[1], jnp.full((pad,), N, jnp.int32)])
    return srcp.reshape(NW, KP, CP), dstp.reshape(NW, KP, CP)


@functools.partial(
    pl.kernel,
    out_type=jax.ShapeDtypeStruct((NC, N, DW), jnp.float32),
    mesh=_mesh,
    compiler_params=pltpu.CompilerParams(use_tc_tiling_on_sc=False),
    scratch_types=[
        pltpu.VMEM((KP, CP), jnp.int32),
        pltpu.VMEM((CP, DW), jnp.float32),
        pltpu.SemaphoreType.DMA,
        pltpu.VMEM_SHARED((NP, DW), jnp.float32),
    ],
)
def _deg_kernel(dstr, zrows, ones, out, dst_all, ones_v, sem_s, deg_sh):
    cid = lax.axis_index("c")
    sid = lax.axis_index("s")
    wid = cid * NS + sid

    @pl.when(sid < NWR)
    def _():
        pltpu.sync_copy(zrows, deg_sh.at[pl.ds(sid * RW, RW)])

    pltpu.sync_copy(ones, ones_v)
    pltpu.sync_copy(dstr.at[wid], dst_all)
    plsc.subcore_barrier()

    def wave(w, carry):
        def fire(k, c):
            pltpu.async_copy(ones_v, deg_sh.at[dst_all.at[k]], sem_s,
                             add=True)
            return c

        lax.fori_loop(w * WAVE_D, (w + 1) * WAVE_D, fire, 0)

        def drain(k, c):
            pltpu.make_async_copy(ones_v, deg_sh.at[dst_all.at[0]],
                                  sem_s).wait()
            return c

        lax.fori_loop(0, WAVE_D, drain, 0)
        return carry

    lax.fori_loop(0, KP // WAVE_D, wave, 0)
    plsc.subcore_barrier()

    @pl.when(sid < NWR)
    def _():
        pltpu.sync_copy(deg_sh.at[pl.ds(sid * RW, RW)],
                        out.at[cid, pl.ds(sid * RW, RW)])


def _make_agg(H):
    @functools.partial(
        pl.kernel,
        out_type=jax.ShapeDtypeStruct((NC, N, H), jnp.float32),
        mesh=_mesh,
        compiler_params=pltpu.CompilerParams(use_tc_tiling_on_sc=False),
        scratch_types=[
            pltpu.VMEM((KP, CP), jnp.int32),
            pltpu.VMEM((KP, CP), jnp.int32),
            pltpu.VMEM((2, WAVE, CP, H), jnp.float32),
            pltpu.SemaphoreType.DMA,
            pltpu.SemaphoreType.DMA,
            pltpu.VMEM_SHARED((NP, H), jnp.float32),
        ],
    )
    def agg_kernel(hs, srcr, dstr, zrows, out, src_all, dst_all, rows,
                   sem_g, sem_s, agg_sh):
        cid = lax.axis_index("c")
        sid = lax.axis_index("s")
        wid = cid * NS + sid

        @pl.when(sid < NWR)
        def _():
            pltpu.sync_copy(zrows, agg_sh.at[pl.ds(sid * RW, RW)])

        pltpu.sync_copy(srcr.at[wid], src_all)
        pltpu.sync_copy(dstr.at[wid], dst_all)
        plsc.subcore_barrier()

        def fire_gathers(w, h):
            def f(j, c):
                pltpu.async_copy(hs.at[src_all.at[w * WAVE + j]],
                                 rows.at[h, j], sem_g)
                return c

            lax.fori_loop(0, WAVE, f, 0)

        def drain(sem, n):
            def f(j, c):
                pltpu.make_async_copy(hs.at[src_all.at[0]], rows.at[0, 0],
                                      sem).wait()
                return c

            lax.fori_loop(0, n, f, 0)

        fire_gathers(0, 0)

        def wave(w, carry):
            h = w % 2
            drain(sem_g, WAVE)

            def fs(j, c):
                pltpu.async_copy(rows.at[h, j],
                                 agg_sh.at[dst_all.at[w * WAVE + j]],
                                 sem_s, add=True)
                return c

            lax.fori_loop(0, WAVE, fs, 0)

            @pl.when(w + 1 < NWAVES)
            def _():
                @pl.when(w >= 1)
                def _():
                    drain(sem_s, WAVE)
                fire_gathers(w + 1, 1 - h)

            return carry

        lax.fori_loop(0, NWAVES, wave, 0)
        drain(sem_s, 2 * WAVE)
        plsc.subcore_barrier()

        @pl.when(sid < NWR)
        def _():
            pltpu.sync_copy(agg_sh.at[pl.ds(sid * RW, RW)],
                            out.at[cid, pl.ds(sid * RW, RW)])

    return agg_kernel


_agg32 = _make_agg(H1)
_agg16 = _make_agg(H2)


def _scale1_body(degp_ref, x_ref, w_ref, dis_ref, hs_ref):
    deg = degp_ref[0, :, 0:1] + degp_ref[1, :, 0:1] + 1.0
    dis = lax.rsqrt(deg)
    dis_ref[...] = dis
    h = jnp.dot(x_ref[...], w_ref[...], preferred_element_type=jnp.float32)
    hs_ref[...] = h * dis


def _scale1(degp, x, W1):
    return pl.pallas_call(
        _scale1_body,
        grid=(N // R,),
        in_specs=[
            pl.BlockSpec((NC, R, DW), lambda i: (0, i, 0)),
            pl.BlockSpec((R, D), lambda i: (i, 0)),
            pl.BlockSpec((D, H1), lambda i: (0, 0)),
        ],
        out_specs=[
            pl.BlockSpec((R, 1), lambda i: (i, 0)),
            pl.BlockSpec((R, H1), lambda i: (i, 0)),
        ],
        out_shape=[
            jax.ShapeDtypeStruct((N, 1), jnp.float32),
            jax.ShapeDtypeStruct((N, H1), jnp.float32),
        ],
    )(degp, x, W1)


def _mid_body(aggp_ref, hs_ref, dis_ref, b1_ref, w2_ref, out_ref):
    dis = dis_ref[...]
    h1 = (aggp_ref[0] + aggp_ref[1] + hs_ref[...]) * dis + b1_ref[...]
    h1 = jnp.maximum(h1, 0.0)
    h2 = jnp.dot(h1, w2_ref[...], preferred_element_type=jnp.float32)
    out_ref[...] = h2 * dis


def _mid(agg1, h1s, dis, b1, W2):
    return pl.pallas_call(
        _mid_body,
        grid=(N // R,),
        in_specs=[
            pl.BlockSpec((NC, R, H1), lambda i: (0, i, 0)),
            pl.BlockSpec((R, H1), lambda i: (i, 0)),
            pl.BlockSpec((R, 1), lambda i: (i, 0)),
            pl.BlockSpec((1, H1), lambda i: (0, 0)),
            pl.BlockSpec((H1, H2), lambda i: (0, 0)),
        ],
        out_specs=pl.BlockSpec((R, H2), lambda i: (i, 0)),
        out_shape=jax.ShapeDtypeStruct((N, H2), jnp.float32),
    )(agg1, h1s, dis, b1, W2)


def _fin_body(aggp_ref, hs_ref, dis_ref, b2_ref, out_ref):
    out_ref[...] = ((aggp_ref[0] + aggp_ref[1] + hs_ref[...])
                    * dis_ref[...] + b2_ref[...])


def _fin(agg2, h2s, dis, b2):
    return pl.pallas_call(
        _fin_body,
        grid=(N // R,),
        in_specs=[
            pl.BlockSpec((NC, R, H2), lambda i: (0, i, 0)),
            pl.BlockSpec((R, H2), lambda i: (i, 0)),
            pl.BlockSpec((R, 1), lambda i: (i, 0)),
            pl.BlockSpec((1, H2), lambda i: (0, 0)),
        ],
        out_specs=pl.BlockSpec((R, H2), lambda i: (i, 0)),
        out_shape=jax.ShapeDtypeStruct((N, H2), jnp.float32),
    )(agg2, h2s, dis, b2)


def kernel(x, edge_index, W1, b1, W2, b2):
    srcr, dstr = _prep_edges(edge_index)
    zdeg = jnp.zeros((RW, DW), jnp.float32)
    z32 = jnp.zeros((RW, H1), jnp.float32)
    z16 = jnp.zeros((RW, H2), jnp.float32)
    ones = jnp.ones((CP, DW), jnp.float32)

    degp = _deg_kernel(dstr, zdeg, ones)
    dis, h1s = _scale1(degp, x, W1)
    agg1 = _agg32(h1s, srcr, dstr, z32)
    h2s = _mid(agg1, h1s, dis, b1.reshape(1, H1), W2)
    agg2 = _agg16(h2s, srcr, dstr, z16)
    return _fin(agg2, h2s, dis, b2.reshape(1, H2))


# speedup vs baseline: 1.6383x; 1.6383x over previous
import functools
import jax
import jax.numpy as jnp
from jax import lax
from jax.experimental import pallas as pl
from jax.experimental.pallas import tpu as pltpu
from jax.experimental.pallas import tpu_sc as plsc

N = 10000
E = 320000
D = 128
H1 = 32
H2 = 16
DW = 8

NC = 2
NS = 16
NW = NC * NS
ET = E // NW
C = 80
K = ET // C
NWR = 10
RW = N // NWR
R = 1000
NBUF = 5
WAVE = 25

_mesh = plsc.VectorSubcoreMesh(core_axis_name="c", subcore_axis_name="s")


@functools.partial(
    pl.kernel,
    out_type=jax.ShapeDtypeStruct((NC, N, DW), jnp.float32),
    mesh=_mesh,
    compiler_params=pltpu.CompilerParams(use_tc_tiling_on_sc=False),
    scratch_types=[
        pltpu.VMEM((K, C), jnp.int32),
        pltpu.VMEM((C, DW), jnp.float32),
        pltpu.SemaphoreType.DMA,
        pltpu.VMEM_SHARED((N, DW), jnp.float32),
    ],
)
def _deg_kernel(dstr, zrows, ones, out, dst_all, ones_v, sem_s, deg_sh):
    cid = lax.axis_index("c")
    sid = lax.axis_index("s")
    wid = cid * NS + sid

    @pl.when(sid < NWR)
    def _():
        pltpu.sync_copy(zrows, deg_sh.at[pl.ds(sid * RW, RW)])

    pltpu.sync_copy(ones, ones_v)
    pltpu.sync_copy(dstr.at[wid], dst_all)
    plsc.subcore_barrier()

    def wave(w, carry):
        def fire(k, c):
            pltpu.async_copy(ones_v, deg_sh.at[dst_all.at[k]], sem_s,
                             add=True)
            return c

        lax.fori_loop(w * WAVE, (w + 1) * WAVE, fire, 0)

        def drain(k, c):
            pltpu.make_async_copy(ones_v, deg_sh.at[dst_all.at[0]],
                                  sem_s).wait()
            return c

        lax.fori_loop(0, WAVE, drain, 0)
        return carry

    lax.fori_loop(0, K // WAVE, wave, 0)
    plsc.subcore_barrier()

    @pl.when(sid < NWR)
    def _():
        pltpu.sync_copy(deg_sh.at[pl.ds(sid * RW, RW)],
                        out.at[cid, pl.ds(sid * RW, RW)])


def _make_agg(H):
    @functools.partial(
        pl.kernel,
        out_type=jax.ShapeDtypeStruct((NC, N, H), jnp.float32),
        mesh=_mesh,
        compiler_params=pltpu.CompilerParams(use_tc_tiling_on_sc=False),
        scratch_types=[
            pltpu.VMEM((K, C), jnp.int32),
            pltpu.VMEM((K, C), jnp.int32),
            pltpu.VMEM((WAVE, C, H), jnp.float32),
            pltpu.SemaphoreType.DMA,
            pltpu.SemaphoreType.DMA,
            pltpu.VMEM_SHARED((N, H), jnp.float32),
        ],
    )
    def agg_kernel(hs, srcr, dstr, zrows, out, src_all, dst_all, rows,
                   sem_g, sem_s, agg_sh):
        cid = lax.axis_index("c")
        sid = lax.axis_index("s")
        wid = cid * NS + sid

        @pl.when(sid < NWR)
        def _():
            pltpu.sync_copy(zrows, agg_sh.at[pl.ds(sid * RW, RW)])

        pltpu.sync_copy(srcr.at[wid], src_all)
        pltpu.sync_copy(dstr.at[wid], dst_all)
        plsc.subcore_barrier()

        def wave(w, carry):
            base = w * WAVE

            def fire_g(j, c):
                pltpu.async_copy(hs.at[src_all.at[base + j]], rows.at[j],
                                 sem_g)
                return c

            lax.fori_loop(0, WAVE, fire_g, 0)

            def drain_g(j, c):
                pltpu.make_async_copy(hs.at[src_all.at[0]], rows.at[0],
                                      sem_g).wait()
                return c

            lax.fori_loop(0, WAVE, drain_g, 0)

            def fire_s(j, c):
                pltpu.async_copy(rows.at[j], agg_sh.at[dst_all.at[base + j]],
                                 sem_s, add=True)
                return c

            lax.fori_loop(0, WAVE, fire_s, 0)

            def drain_s(j, c):
                pltpu.make_async_copy(rows.at[0], agg_sh.at[dst_all.at[0]],
                                      sem_s).wait()
                return c

            lax.fori_loop(0, WAVE, drain_s, 0)
            return carry

        lax.fori_loop(0, K // WAVE, wave, 0)
        plsc.subcore_barrier()

        @pl.when(sid < NWR)
        def _():
            pltpu.sync_copy(agg_sh.at[pl.ds(sid * RW, RW)],
                            out.at[cid, pl.ds(sid * RW, RW)])

    return agg_kernel


_agg32 = _make_agg(H1)
_agg16 = _make_agg(H2)


def _scale1_body(degp_ref, x_ref, w_ref, dis_ref, hs_ref):
    deg = degp_ref[0, :, 0:1] + degp_ref[1, :, 0:1] + 1.0
    dis = lax.rsqrt(deg)
    dis_ref[...] = dis
    h = jnp.dot(x_ref[...], w_ref[...], preferred_element_type=jnp.float32)
    hs_ref[...] = h * dis


def _scale1(degp, x, W1):
    return pl.pallas_call(
        _scale1_body,
        grid=(N // R,),
        in_specs=[
            pl.BlockSpec((NC, R, DW), lambda i: (0, i, 0)),
            pl.BlockSpec((R, D), lambda i: (i, 0)),
            pl.BlockSpec((D, H1), lambda i: (0, 0)),
        ],
        out_specs=[
            pl.BlockSpec((R, 1), lambda i: (i, 0)),
            pl.BlockSpec((R, H1), lambda i: (i, 0)),
        ],
        out_shape=[
            jax.ShapeDtypeStruct((N, 1), jnp.float32),
            jax.ShapeDtypeStruct((N, H1), jnp.float32),
        ],
    )(degp, x, W1)


def _mid_body(aggp_ref, hs_ref, dis_ref, b1_ref, w2_ref, out_ref):
    dis = dis_ref[...]
    h1 = (aggp_ref[0] + aggp_ref[1] + hs_ref[...]) * dis + b1_ref[...]
    h1 = jnp.maximum(h1, 0.0)
    h2 = jnp.dot(h1, w2_ref[...], preferred_element_type=jnp.float32)
    out_ref[...] = h2 * dis


def _mid(agg1, h1s, dis, b1, W2):
    return pl.pallas_call(
        _mid_body,
        grid=(N // R,),
        in_specs=[
            pl.BlockSpec((NC, R, H1), lambda i: (0, i, 0)),
            pl.BlockSpec((R, H1), lambda i: (i, 0)),
            pl.BlockSpec((R, 1), lambda i: (i, 0)),
            pl.BlockSpec((1, H1), lambda i: (0, 0)),
            pl.BlockSpec((H1, H2), lambda i: (0, 0)),
        ],
        out_specs=pl.BlockSpec((R, H2), lambda i: (i, 0)),
        out_shape=jax.ShapeDtypeStruct((N, H2), jnp.float32),
    )(agg1, h1s, dis, b1, W2)


def _fin_body(aggp_ref, hs_ref, dis_ref, b2_ref, out_ref):
    out_ref[...] = ((aggp_ref[0] + aggp_ref[1] + hs_ref[...])
                    * dis_ref[...] + b2_ref[...])


def _fin(agg2, h2s, dis, b2):
    return pl.pallas_call(
        _fin_body,
        grid=(N // R,),
        in_specs=[
            pl.BlockSpec((NC, R, H2), lambda i: (0, i, 0)),
            pl.BlockSpec((R, H2), lambda i: (i, 0)),
            pl.BlockSpec((R, 1), lambda i: (i, 0)),
            pl.BlockSpec((1, H2), lambda i: (0, 0)),
        ],
        out_specs=pl.BlockSpec((R, H2), lambda i: (i, 0)),
        out_shape=jax.ShapeDtypeStruct((N, H2), jnp.float32),
    )(agg2, h2s, dis, b2)


def kernel(x, edge_index, W1, b1, W2, b2):
    srcr = edge_index[0].reshape(NW, K, C)
    dstr = edge_index[1].reshape(NW, K, C)
    zdeg = jnp.zeros((RW, DW), jnp.float32)
    z32 = jnp.zeros((RW, H1), jnp.float32)
    z16 = jnp.zeros((RW, H2), jnp.float32)
    ones = jnp.ones((C, DW), jnp.float32)

    degp = _deg_kernel(dstr, zdeg, ones)
    dis, h1s = _scale1(degp, x, W1)
    agg1 = _agg32(h1s, srcr, dstr, z32)
    h2s = _mid(agg1, h1s, dis, b1.reshape(1, H1), W2)
    agg2 = _agg16(h2s, srcr, dstr, z16)
    return _fin(agg2, h2s, dis, b2.reshape(1, H2))
